# baseline (speedup 1.0000x reference)
import functools

import jax
import jax.numpy as jnp
from jax import lax
from jax.experimental import pallas as pl
from jax.experimental.pallas import tpu as pltpu
from jax.experimental.pallas import tpu_sc as plsc

VOCAB = 1000000
D = 64
T = 16384
SCALE = 8.0

_INFO = plsc.get_sparse_core_info()
NC = _INFO.num_cores
NS = _INFO.num_subcores
NW = NC * NS
NT = T // NW
CHUNK = 128
N_CHUNKS = NT // CHUNK

_mesh = plsc.VectorSubcoreMesh(core_axis_name="c", subcore_axis_name="s")


@functools.partial(
    pl.kernel,
    mesh=_mesh,
    out_type=jax.ShapeDtypeStruct((T, 2 * D), jnp.float32),
    scratch_types=[
        pltpu.VMEM((NT,), jnp.int32),
        pltpu.VMEM((NT,), jnp.int32),
        pltpu.VMEM((NT, 2 * D), jnp.float32),
        pltpu.SemaphoreType.DMA,
    ],
)
def _gather_pairs(table2_hbm, idx_hbm, out_hbm, idx_v, pidx_v, rows_v, sem):
    wid = lax.axis_index("s") * NC + lax.axis_index("c")
    tbase = wid * NT

    pltpu.sync_copy(idx_hbm.at[pl.ds(tbase, NT)], idx_v)

    for j in range(NT // 16):
        sl = pl.ds(j * 16, 16)
        pidx_v[sl] = jax.lax.shift_right_logical(idx_v[sl], 1)

    copies = []
    for j in range(N_CHUNKS):
        copies.append(
            pltpu.async_copy(
                table2_hbm.at[pidx_v.at[pl.ds(j * CHUNK, CHUNK)]],
                rows_v.at[pl.ds(j * CHUNK, CHUNK)],
                sem,
            )
        )
    for c in copies:
        c.wait()

    pltpu.sync_copy(rows_v, out_hbm.at[pl.ds(tbase, NT)])


BT = 512


def _select_body(idx_ref, pair_ref, o_ref):
    odd = (idx_ref[...] & 1) == 1
    lo = pair_ref[:, :D]
    hi = pair_ref[:, D:]
    o_ref[...] = jnp.where(odd, hi, lo) * SCALE


_select = pl.pallas_call(
    _select_body,
    grid=(T // BT,),
    in_specs=[
        pl.BlockSpec((BT, 1), lambda i: (i, 0)),
        pl.BlockSpec((BT, 2 * D), lambda i: (i, 0)),
    ],
    out_specs=pl.BlockSpec((BT, D), lambda i: (i, 0)),
    out_shape=jax.ShapeDtypeStruct((T, D), jnp.float32),
)


def kernel(x, input_embedding_table_VD):
    xi = x.astype(jnp.int32)
    table2 = input_embedding_table_VD.reshape(VOCAB // 2, 2 * D)
    pair_rows = _gather_pairs(table2, xi)
    return _select(xi.reshape(T, 1), pair_rows)
